# Initial kernel scaffold; baseline (speedup 1.0000x reference)
import jax
import jax.numpy as jnp
from jax.experimental import pallas as pl


def kernel(inputs, sparse_ancestors, sparse_ancestors_values, w, b):
    raise NotImplementedError("write your pallas kernel here")


# speedup vs baseline: 3.8835x; 3.8835x over previous
import functools

import jax
import jax.numpy as jnp
from jax import lax
from jax.experimental import pallas as pl
from jax.experimental.pallas import tpu as pltpu
from jax.experimental.pallas import tpu_sc as plsc

N_CONCEPTS = 10000
NNZ = 320000
D = 128
B = 256

NC = 2
NS = 16
NW = NC * NS
L = 16
CHUNK = 128

NPAD = 10240
NNZ_PAD = ((NNZ + NW * CHUNK - 1) // (NW * CHUNK)) * (NW * CHUNK)
EPT = NNZ_PAD // NW
CHUNKS_PER_TILE = EPT // CHUNK
ZROWS = NPAD // NS

_GDN = lax.GatherDimensionNumbers(
    offset_dims=(), collapsed_slice_dims=(0,), start_index_map=(0,))


def _bcast_lane(v, k):
    idx = jnp.full((L, 1), k, jnp.int32)
    return lax.gather(v, idx, _GDN, (1,),
                      mode=lax.GatherScatterMode.PROMISE_IN_BOUNDS)


def _sc_body(rows_hbm, cols_hbm, vals_hbm, w_hbm, out_hbm,
             colb, rowb, valb, gbuf, acc, gsem):
    c = lax.axis_index("c")
    s = lax.axis_index("s")
    tid = c * NS + s
    base = tid * EPT

    zeros16 = jnp.zeros((L,), jnp.float32)
    def zero_row(r, _):
        for j in range(D // L):
            gbuf[0, r, pl.ds(j * L, L)] = zeros16
        return 0
    lax.fori_loop(0, CHUNK, zero_row, 0)
    for k in range(ZROWS // CHUNK):
        pltpu.sync_copy(gbuf.at[0], acc.at[pl.ds(s * ZROWS + k * CHUNK, CHUNK)])
    plsc.subcore_barrier()

    def chunk_body(i, _):
        off = base + i * CHUNK
        pltpu.sync_copy(cols_hbm.at[pl.ds(off, CHUNK)], colb.at[0])
        pltpu.sync_copy(rows_hbm.at[pl.ds(off, CHUNK)], rowb.at[0])
        pltpu.sync_copy(vals_hbm.at[pl.ds(off, CHUNK)], valb.at[0])
        pltpu.async_copy(w_hbm.at[colb.at[0]], gbuf.at[0], gsem).wait()

        def group(gi, _):
            v16 = valb[0, pl.ds(gi * L, L)]
            for k in range(L):
                e = gi * L + k
                bc = _bcast_lane(v16, k)
                for j in range(D // L):
                    sl = pl.ds(j * L, L)
                    gbuf[0, e, sl] = gbuf[0, e, sl] * bc
            return 0
        lax.fori_loop(0, CHUNK // L, group, 0)

        pltpu.sync_copy(gbuf.at[0], acc.at[rowb.at[0]], add=True)
        return 0

    lax.fori_loop(0, CHUNKS_PER_TILE, chunk_body, 0)
    plsc.subcore_barrier()

    pltpu.sync_copy(acc.at[pl.ds(s * ZROWS, ZROWS)],
                    out_hbm.at[c, pl.ds(s * ZROWS, ZROWS)])


_sc_aggregate = functools.partial(
    pl.kernel,
    out_type=jax.ShapeDtypeStruct((NC, NPAD, D), jnp.float32),
    mesh=plsc.VectorSubcoreMesh(core_axis_name="c", subcore_axis_name="s",
                                num_cores=NC, num_subcores=NS),
    scratch_types=[
        pltpu.VMEM((2, CHUNK), jnp.int32),
        pltpu.VMEM((2, CHUNK), jnp.int32),
        pltpu.VMEM((2, CHUNK), jnp.float32),
        pltpu.VMEM((2, CHUNK, D), jnp.float32),
        pltpu.VMEM_SHARED((NPAD, D), jnp.float32),
        pltpu.SemaphoreType.DMA,
    ],
)(_sc_body)


def _tc_matmul_body(x_ref, a0_ref, a1_ref, b_ref, o_ref):
    aw = a0_ref[...] + a1_ref[...]
    acc = lax.dot_general(x_ref[...], aw, (((1,), (1,)), ((), ())),
                          preferred_element_type=jnp.float32)
    o_ref[...] = acc + b_ref[...][None, :]


def kernel(inputs, sparse_ancestors, sparse_ancestors_values, w, b):
    rows = sparse_ancestors[:, 0]
    cols = sparse_ancestors[:, 1]
    pad = NNZ_PAD - NNZ
    rows = jnp.pad(rows, (0, pad))
    cols = jnp.pad(cols, (0, pad))
    vals = jnp.pad(sparse_ancestors_values, (0, pad))

    parts = _sc_aggregate(rows, cols, vals, w)

    b_pad = jnp.pad(b, (0, NPAD - N_CONCEPTS))
    out = pl.pallas_call(
        _tc_matmul_body,
        out_shape=jax.ShapeDtypeStruct((B, NPAD), jnp.float32),
    )(inputs, parts[0], parts[1], b_pad)
    return out[:, :N_CONCEPTS]
